# baseline (speedup 1.0000x reference)
import functools

import jax
import jax.numpy as jnp
from jax import lax
from jax.experimental import pallas as pl
from jax.experimental.pallas import tpu as pltpu
from jax.experimental.pallas import tpu_sc as plsc

_B = 4
_T = 4096
_D = 1024
_NW = 32
_TW = _T // _NW
_C = 16
_NTC = _TW // _C
_NJOB = _B * _NTC
_NBUF = 5
_LOOK = 3
_LANES = 16
_SL = _D // _LANES


def _emb_body(ids_hbm, tok_hbm, pos_hbm, out_hbm,
              idx_v, pos0, pos1, tk0, tk1, tk2, tk3, tk4,
              sp0, sp1, sg0, sg1, sg2, sg3, sg4, ss0, ss1, ss2, ss3, ss4):
    wid = lax.axis_index("s") * 2 + lax.axis_index("c")
    t0 = wid * _TW

    toks = (tk0, tk1, tk2, tk3, tk4)
    sgs = (sg0, sg1, sg2, sg3, sg4)
    sss = (ss0, ss1, ss2, ss3, ss4)
    poss = (pos0, pos1)
    sps = (sp0, sp1)

    for b in range(_B):
        pltpu.sync_copy(ids_hbm.at[pl.ds(b * _T + t0, _TW)],
                        idx_v.at[pl.ds(b * _TW, _TW)])

    def fire_gather(i):
        b, tc = i % _B, i // _B
        ioff = b * _TW + tc * _C
        pltpu.async_copy(tok_hbm.at[idx_v.at[pl.ds(ioff, _C)]],
                         toks[i % _NBUF], sgs[i % _NBUF])

    def fire_pos(p):
        pltpu.async_copy(pos_hbm.at[pl.ds(t0 + p * _C, _C)],
                         poss[p % 2], sps[p % 2])

    def wait_store(i):
        pltpu.make_async_copy(toks[i % _NBUF], out_hbm.at[pl.ds(0, _C)],
                              sss[i % _NBUF]).wait()

    fire_pos(0)
    for i in range(_LOOK):
        fire_gather(i)

    for i in range(_NJOB):
        k = i % _NBUF
        tok = toks[k]

        if i + _LOOK <= _NJOB - 1:
            if i + _LOOK - _NBUF >= 0:
                wait_store(i + _LOOK - _NBUF)
            fire_gather(i + _LOOK)

        if i % _B == 0:
            p = i // _B
            pltpu.make_async_copy(pos_hbm.at[pl.ds(0, _C)],
                                  poss[p % 2], sps[p % 2]).wait()
            if p + 1 <= _NTC - 1:
                fire_pos(p + 1)

        pltpu.make_async_copy(tok_hbm.at[idx_v.at[pl.ds(0, _C)]],
                              tok, sgs[k]).wait()
        posb = poss[(i // _B) % 2]

        def add_half(it, _):
            r = lax.div(it, 2)
            base = lax.rem(it, 2) * (_SL // 2 * _LANES)
            for j in range(_SL // 2):
                sl = pl.ds(base + j * _LANES, _LANES)
                plsc.addupdate(tok.at[r, sl], posb[r, sl])
            return 0

        lax.fori_loop(0, 2 * _C, add_half, 0)

        b, tc = i % _B, i // _B
        row = b * _T + t0 + tc * _C
        pltpu.async_copy(tok, out_hbm.at[pl.ds(row, _C)], sss[k])

    for i in range(_NJOB - _NBUF, _NJOB):
        wait_store(i)


@jax.jit
def _emb(ids_flat, token_emb, pos_emb):
    mesh = plsc.VectorSubcoreMesh(core_axis_name="c", subcore_axis_name="s")
    call = functools.partial(
        pl.kernel,
        mesh=mesh,
        out_type=jax.ShapeDtypeStruct((_B * _T, _D), jnp.float32),
        scratch_types=(
            [pltpu.VMEM((_B * _TW,), jnp.int32)]
            + [pltpu.VMEM((_C, _D), jnp.float32)] * (2 + _NBUF)
            + [pltpu.SemaphoreType.DMA] * (2 + 2 * _NBUF)
        ),
    )(_emb_body)
    return call(ids_flat, token_emb, pos_emb)


def kernel(input_ids, token_emb, pos_emb):
    ids_flat = input_ids.astype(jnp.int32).reshape(-1)
    out = _emb(ids_flat, token_emb, pos_emb)
    return out.reshape(_B, _T, _D)
